# baseline (speedup 1.0000x reference)
import functools

import jax
import jax.numpy as jnp
import numpy as np
from jax.experimental import pallas as pl
from jax.experimental.pallas import tpu as pltpu


def _fused_kernel(e_cur, e_prev, eall_ref, x_ref, wax_ref, wbx_ref,
                  pmat_ref, ssel_ref, bp_ref, lwsel_ref, lb_ref, out_ref,
                  u_ref, v_ref, p_ref, *, n_rows, blk_r, ncols, n_blocks):
    i = pl.program_id(0)

    @pl.when(i == 0)
    def _prep():
        nb = x_ref.shape[0]
        u_acc = jnp.dot(x_ref[0], wax_ref[0],
                        preferred_element_type=jnp.float32)
        v_acc = jnp.dot(x_ref[0], wbx_ref[0],
                        preferred_element_type=jnp.float32)
        for b in range(1, nb):
            u_acc = u_acc + jnp.dot(x_ref[b], wax_ref[b],
                                    preferred_element_type=jnp.float32)
            v_acc = v_acc + jnp.dot(x_ref[b], wbx_ref[b],
                                    preferred_element_type=jnp.float32)
        u_ref[0:n_rows, :] = u_acc
        v_ref[:, 0:ncols] = v_acc
        v_ref[:, ncols:ncols + 1] = jnp.ones((n_rows, 1), jnp.float32)

    @pl.when(i < n_blocks)
    def _stage_a():
        a = jax.lax.dot_general(e_cur[...], eall_ref[...],
                                (((1,), (1,)), ((), ())),
                                preferred_element_type=jnp.float32)
        p_ref[pl.ds((i % 2) * blk_r, blk_r), :] = jnp.exp(
            jnp.minimum(jnp.maximum(a, 0.0), 85.0))

    @pl.when(i > 0)
    def _stage_b():
        j = i - 1
        eb = e_prev[...]
        pb = p_ref[pl.ds((j % 2) * blk_r, blk_r), :]
        pv = jnp.dot(pb, v_ref[...], preferred_element_type=jnp.float32)
        inv = 1.0 / pv[:, ncols:ncols + 1]

        t = u_ref[pl.ds(j * blk_r, blk_r), :] + pv[:, 0:ncols] * inv
        emul = jnp.dot(eb, pmat_ref[...], preferred_element_type=jnp.float32)
        gu = (jnp.dot(t * emul, ssel_ref[...],
                      preferred_element_type=jnp.float32)
              + jnp.dot(eb, bp_ref[...], preferred_element_type=jnp.float32))
        r = jax.nn.sigmoid(gu[:, 0:8])
        h = jnp.tanh(gu[:, 8:16])
        y = jnp.maximum((1.0 - r) * h, 0.0)
        yo = (jnp.dot(y, lwsel_ref[...], preferred_element_type=jnp.float32)
              + lb_ref[0:1, 0:1])
        out_ref[...] = yo.T


def kernel(x, e, gate_weights_pool, gate_bias_pool, update_weights_pool,
           update_bias_pool, linear_w, linear_b):
    B, N, C = x.shape
    D = e.shape[1]
    R = 384
    ng = pl.cdiv(N, R)

    gw = gate_weights_pool[:, :, :C, :]
    uw = update_weights_pool[:, :, :C, :]
    wa1 = jnp.concatenate([
        jnp.transpose(gw[:, 0], (1, 0, 2)).reshape(C, 4 * D),
        jnp.transpose(uw[:, 0], (1, 0, 2)).reshape(C, 2 * D),
    ], axis=1)
    wb1 = jnp.concatenate([
        jnp.transpose(gw[:, 1], (1, 0, 2)).reshape(C, 4 * D),
        jnp.transpose(uw[:, 1], (1, 0, 2)).reshape(C, 2 * D),
    ], axis=1)
    eyeb = jnp.eye(B, dtype=jnp.float32)
    wax = jnp.kron(eyeb, wa1).reshape(B, C, 24 * B)
    wbx = jnp.kron(eyeb, wb1).reshape(B, C, 24 * B)

    pm1 = np.zeros((D, 24), np.float32)
    for d in range(D):
        pm1[d, 4 * d:4 * d + 4] = 1.0
        pm1[d, 16 + 2 * d:16 + 2 * d + 2] = 1.0
    pmat = jnp.asarray(np.tile(pm1, (1, B)))

    ss1 = np.zeros((24, 16), np.float32)
    for d in range(D):
        for j in range(2):
            ss1[4 * d + 2 + j, j] = 1.0
            ss1[16 + 2 * d + j, 8 + j] = 1.0
    ssel_np = np.zeros((B * 24, 16), np.float32)
    for b in range(B):
        ssel_np[b * 24:(b + 1) * 24, 2 * b:2 * b + 2] = ss1[:, 0:2]
        ssel_np[b * 24:(b + 1) * 24, 8 + 2 * b:8 + 2 * b + 2] = ss1[:, 8:10]
    ssel = jnp.asarray(ssel_np)

    bp = jnp.concatenate([gate_bias_pool[:, 2:4]] * B
                         + [update_bias_pool] * B, axis=1)

    lwsel = jnp.kron(eyeb, linear_w.T)
    lb2 = linear_b.reshape(1, 1)

    nc = 24 * B
    y2 = pl.pallas_call(
        functools.partial(_fused_kernel, n_rows=N, blk_r=R, ncols=nc,
                          n_blocks=ng),
        grid=(ng + 1,),
        in_specs=[
            pl.BlockSpec((R, D), lambda i, m=ng - 1: (jnp.minimum(i, m), 0)),
            pl.BlockSpec((R, D), lambda i: (jnp.maximum(i - 1, 0), 0)),
            pl.BlockSpec((N, D), lambda i: (0, 0)),
            pl.BlockSpec((B, N, C), lambda i: (0, 0, 0)),
            pl.BlockSpec((B, C, nc), lambda i: (0, 0, 0)),
            pl.BlockSpec((B, C, nc), lambda i: (0, 0, 0)),
            pl.BlockSpec((D, nc), lambda i: (0, 0)),
            pl.BlockSpec((nc, 4 * B), lambda i: (0, 0)),
            pl.BlockSpec((D, 4 * B), lambda i: (0, 0)),
            pl.BlockSpec((2 * B, B), lambda i: (0, 0)),
            pl.BlockSpec((1, 1), lambda i: (0, 0)),
        ],
        out_specs=pl.BlockSpec((B, R), lambda i: (0, jnp.maximum(i - 1, 0))),
        out_shape=jax.ShapeDtypeStruct((B, N), jnp.float32),
        scratch_shapes=[
            pltpu.VMEM((ng * R, nc), jnp.float32),
            pltpu.VMEM((N, 128), jnp.float32),
            pltpu.VMEM((2 * R, N), jnp.float32),
        ],
        compiler_params=pltpu.CompilerParams(
            dimension_semantics=("arbitrary",),
        ),
    )(e, e, e, x, wax, wbx, pmat, ssel, bp, lwsel, lb2)

    return y2[:, :, None]


# speedup vs baseline: 1.0464x; 1.0464x over previous
import functools

import jax
import jax.numpy as jnp
import numpy as np
from jax.experimental import pallas as pl
from jax.experimental.pallas import tpu as pltpu


def _pipelined_step(i, e_cur, e_prev, eall_ref, pmat_ref, ssel_ref, bp_ref,
                    lwsel_ref, lb_ref, out_ref, u_ref, v_ref,
                    p_wr_ref, p_rd_ref, *, blk_r, ncols):
    a = jax.lax.dot_general(e_cur[...], eall_ref[...],
                            (((1,), (1,)), ((), ())),
                            preferred_element_type=jnp.float32)
    p_wr_ref[...] = jnp.exp(jnp.minimum(jnp.maximum(a, 0.0), 85.0))

    j = jnp.maximum(i - 1, 0)
    eb = e_prev[...]
    pv = jnp.dot(p_rd_ref[...], v_ref[...],
                 preferred_element_type=jnp.float32)
    inv = 1.0 / pv[:, ncols:ncols + 1]

    t = u_ref[pl.ds(j * blk_r, blk_r), :] + pv[:, 0:ncols] * inv
    emul = jnp.dot(eb, pmat_ref[...], preferred_element_type=jnp.float32)
    gu = (jnp.dot(t * emul, ssel_ref[...],
                  preferred_element_type=jnp.float32)
          + jnp.dot(eb, bp_ref[...], preferred_element_type=jnp.float32))
    r = jax.nn.sigmoid(gu[:, 0:8])
    h = jnp.tanh(gu[:, 8:16])
    y = jnp.maximum((1.0 - r) * h, 0.0)
    yo = (jnp.dot(y, lwsel_ref[...], preferred_element_type=jnp.float32)
          + lb_ref[0:1, 0:1])
    out_ref[...] = yo.T


def _fused_kernel(e_cur, e_prev, eall_ref, x_ref, wax_ref, wbx_ref,
                  pmat_ref, ssel_ref, bp_ref, lwsel_ref, lb_ref, out_ref,
                  u_ref, v_ref, p0_ref, p1_ref, *, n_rows, blk_r, ncols,
                  n_blocks):
    i = pl.program_id(0)

    @pl.when(i == 0)
    def _prep():
        nb = x_ref.shape[0]
        u_acc = jnp.dot(x_ref[0], wax_ref[0],
                        preferred_element_type=jnp.float32)
        v_acc = jnp.dot(x_ref[0], wbx_ref[0],
                        preferred_element_type=jnp.float32)
        for b in range(1, nb):
            u_acc = u_acc + jnp.dot(x_ref[b], wax_ref[b],
                                    preferred_element_type=jnp.float32)
            v_acc = v_acc + jnp.dot(x_ref[b], wbx_ref[b],
                                    preferred_element_type=jnp.float32)
        u_ref[0:n_rows, :] = u_acc
        v_ref[:, 0:ncols] = v_acc
        v_ref[:, ncols:ncols + 1] = jnp.ones((n_rows, 1), jnp.float32)

    step = functools.partial(
        _pipelined_step, i, e_cur, e_prev, eall_ref, pmat_ref, ssel_ref,
        bp_ref, lwsel_ref, lb_ref, out_ref, u_ref, v_ref,
        blk_r=blk_r, ncols=ncols)

    @pl.when(i % 2 == 0)
    def _even():
        step(p0_ref, p1_ref)

    @pl.when(i % 2 == 1)
    def _odd():
        step(p1_ref, p0_ref)


def kernel(x, e, gate_weights_pool, gate_bias_pool, update_weights_pool,
           update_bias_pool, linear_w, linear_b):
    B, N, C = x.shape
    D = e.shape[1]
    R = 512
    ng = pl.cdiv(N, R)

    gw = gate_weights_pool[:, :, :C, :]
    uw = update_weights_pool[:, :, :C, :]
    wa1 = jnp.concatenate([
        jnp.transpose(gw[:, 0], (1, 0, 2)).reshape(C, 4 * D),
        jnp.transpose(uw[:, 0], (1, 0, 2)).reshape(C, 2 * D),
    ], axis=1)
    wb1 = jnp.concatenate([
        jnp.transpose(gw[:, 1], (1, 0, 2)).reshape(C, 4 * D),
        jnp.transpose(uw[:, 1], (1, 0, 2)).reshape(C, 2 * D),
    ], axis=1)
    eyeb = jnp.eye(B, dtype=jnp.float32)
    wax = jnp.kron(eyeb, wa1).reshape(B, C, 24 * B)
    wbx = jnp.kron(eyeb, wb1).reshape(B, C, 24 * B)

    pm1 = np.zeros((D, 24), np.float32)
    for d in range(D):
        pm1[d, 4 * d:4 * d + 4] = 1.0
        pm1[d, 16 + 2 * d:16 + 2 * d + 2] = 1.0
    pmat = jnp.asarray(np.tile(pm1, (1, B)))

    ss1 = np.zeros((24, 16), np.float32)
    for d in range(D):
        for j in range(2):
            ss1[4 * d + 2 + j, j] = 1.0
            ss1[16 + 2 * d + j, 8 + j] = 1.0
    ssel_np = np.zeros((B * 24, 16), np.float32)
    for b in range(B):
        ssel_np[b * 24:(b + 1) * 24, 2 * b:2 * b + 2] = ss1[:, 0:2]
        ssel_np[b * 24:(b + 1) * 24, 8 + 2 * b:8 + 2 * b + 2] = ss1[:, 8:10]
    ssel = jnp.asarray(ssel_np)

    bp = jnp.concatenate([gate_bias_pool[:, 2:4]] * B
                         + [update_bias_pool] * B, axis=1)

    lwsel = jnp.kron(eyeb, linear_w.T)
    lb2 = linear_b.reshape(1, 1)

    nc = 24 * B
    y2 = pl.pallas_call(
        functools.partial(_fused_kernel, n_rows=N, blk_r=R, ncols=nc,
                          n_blocks=ng),
        grid=(ng + 1,),
        in_specs=[
            pl.BlockSpec((R, D), lambda i, m=ng - 1: (jnp.minimum(i, m), 0)),
            pl.BlockSpec((R, D), lambda i: (jnp.maximum(i - 1, 0), 0)),
            pl.BlockSpec((N, D), lambda i: (0, 0)),
            pl.BlockSpec((B, N, C), lambda i: (0, 0, 0)),
            pl.BlockSpec((B, C, nc), lambda i: (0, 0, 0)),
            pl.BlockSpec((B, C, nc), lambda i: (0, 0, 0)),
            pl.BlockSpec((D, nc), lambda i: (0, 0)),
            pl.BlockSpec((nc, 4 * B), lambda i: (0, 0)),
            pl.BlockSpec((D, 4 * B), lambda i: (0, 0)),
            pl.BlockSpec((2 * B, B), lambda i: (0, 0)),
            pl.BlockSpec((1, 1), lambda i: (0, 0)),
        ],
        out_specs=pl.BlockSpec((B, R), lambda i: (0, jnp.maximum(i - 1, 0))),
        out_shape=jax.ShapeDtypeStruct((B, N), jnp.float32),
        scratch_shapes=[
            pltpu.VMEM((ng * R, nc), jnp.float32),
            pltpu.VMEM((N, 128), jnp.float32),
            pltpu.VMEM((R, N), jnp.float32),
            pltpu.VMEM((R, N), jnp.float32),
        ],
        compiler_params=pltpu.CompilerParams(
            dimension_semantics=("arbitrary",),
        ),
    )(e, e, e, x, wax, wbx, pmat, ssel, bp, lwsel, lb2)

    return y2[:, :, None]


# speedup vs baseline: 1.0678x; 1.0205x over previous
import functools

import jax
import jax.numpy as jnp
import numpy as np
from jax.experimental import pallas as pl
from jax.experimental.pallas import tpu as pltpu


def _fused_kernel(e_blk, eall_ref, x_ref, gf_ref, uf_ref, gbp_ref, ubp_ref,
                  lw_ref, lb_ref, pmat_ref, ssel_ref, posg_ref, posu_ref,
                  pb_ref, g1_ref, g2_ref, t2_ref, sum2_ref, out_ref,
                  u_ref, v_ref, *, n_rows, blk_r, ncols, cin, emb_dim):
    i = pl.program_id(0)

    @pl.when(i == 0)
    def _prep():
        cw = gf_ref.shape[0] // (2 * emb_dim)
        w_k = []
        for k in range(2):
            acc = None
            for d in range(emb_dim):
                off = (2 * d + k) * cw
                term = (jnp.dot(gf_ref[off:off + cin, :], posg_ref[d],
                                preferred_element_type=jnp.float32)
                        + jnp.dot(uf_ref[off:off + cin, :], posu_ref[d],
                                  preferred_element_type=jnp.float32))
                acc = term if acc is None else acc + term
            w_k.append(acc)
        nb = x_ref.shape[0]
        u_acc = None
        v_acc = None
        for b in range(nb):
            wab = jnp.dot(w_k[0], pb_ref[b], preferred_element_type=jnp.float32)
            wbb = jnp.dot(w_k[1], pb_ref[b], preferred_element_type=jnp.float32)
            du = jnp.dot(x_ref[b], wab, preferred_element_type=jnp.float32)
            dv = jnp.dot(x_ref[b], wbb, preferred_element_type=jnp.float32)
            u_acc = du if u_acc is None else u_acc + du
            v_acc = dv if v_acc is None else v_acc + dv
        u_ref[0:n_rows, :] = u_acc
        v_ref[:, 0:ncols] = v_acc
        v_ref[:, ncols:ncols + 1] = jnp.ones((n_rows, 1), jnp.float32)

    eb = e_blk[...]
    a = jax.lax.dot_general(eb, eall_ref[...],
                            (((1,), (1,)), ((), ())),
                            preferred_element_type=jnp.float32)
    p = jnp.exp(jnp.minimum(jnp.maximum(a, 0.0), 85.0))
    pv = jnp.dot(p, v_ref[...], preferred_element_type=jnp.float32)
    inv = 1.0 / pv[:, ncols:ncols + 1]

    t = u_ref[pl.ds(i * blk_r, blk_r), :] + pv[:, 0:ncols] * inv
    emul = jnp.dot(eb, pmat_ref[...], preferred_element_type=jnp.float32)
    gu = (jnp.dot(t * emul, ssel_ref[...], preferred_element_type=jnp.float32)
          + jnp.dot(jnp.dot(eb, gbp_ref[...],
                            preferred_element_type=jnp.float32),
                    g1_ref[...], preferred_element_type=jnp.float32)
          + jnp.dot(jnp.dot(eb, ubp_ref[...],
                            preferred_element_type=jnp.float32),
                    g2_ref[...], preferred_element_type=jnp.float32))
    r = jax.nn.sigmoid(gu[:, 0:8])
    h = jnp.tanh(gu[:, 8:16])
    y = jnp.maximum((1.0 - r) * h, 0.0)
    lwt = jnp.dot(lw_ref[...], t2_ref[...],
                  preferred_element_type=jnp.float32)
    yo = (jnp.dot(y * lwt, sum2_ref[...], preferred_element_type=jnp.float32)
          + lb_ref[0:1, 0:1])
    out_ref[...] = yo.T


def kernel(x, e, gate_weights_pool, gate_bias_pool, update_weights_pool,
           update_bias_pool, linear_w, linear_b):
    B, N, C = x.shape
    D = e.shape[1]
    Cw = gate_weights_pool.shape[2]
    R = 512
    ng = pl.cdiv(N, R)
    nc = 24 * B

    gflat = gate_weights_pool.reshape(D * 2 * Cw, 4)
    uflat = update_weights_pool.reshape(D * 2 * Cw, 2)
    lb2 = linear_b.reshape(1, 1)

    posg = np.zeros((D, 4, 24), np.float32)
    posu = np.zeros((D, 2, 24), np.float32)
    for d in range(D):
        for o in range(4):
            posg[d, o, 4 * d + o] = 1.0
        for o in range(2):
            posu[d, o, 16 + 2 * d + o] = 1.0
    pbmat = np.zeros((B, 24, nc), np.float32)
    for b in range(B):
        for c in range(24):
            pbmat[b, c, 24 * b + c] = 1.0
    pm1 = np.zeros((D, 24), np.float32)
    for d in range(D):
        pm1[d, 4 * d:4 * d + 4] = 1.0
        pm1[d, 16 + 2 * d:16 + 2 * d + 2] = 1.0
    pmat = np.tile(pm1, (1, B))
    ss1 = np.zeros((24, 16), np.float32)
    for d in range(D):
        for j in range(2):
            ss1[4 * d + 2 + j, j] = 1.0
            ss1[16 + 2 * d + j, 8 + j] = 1.0
    ssel = np.zeros((nc, 16), np.float32)
    for b in range(B):
        ssel[b * 24:(b + 1) * 24, 2 * b:2 * b + 2] = ss1[:, 0:2]
        ssel[b * 24:(b + 1) * 24, 8 + 2 * b:8 + 2 * b + 2] = ss1[:, 8:10]
    g1 = np.zeros((4, 16), np.float32)
    g2 = np.zeros((2, 16), np.float32)
    for b in range(B):
        for j in range(2):
            g1[2 + j, 2 * b + j] = 1.0
            g2[j, 8 + 2 * b + j] = 1.0
    t2 = np.zeros((2, 2 * B), np.float32)
    for b in range(B):
        for j in range(2):
            t2[j, 2 * b + j] = 1.0
    sum2 = np.zeros((2 * B, B), np.float32)
    for b in range(B):
        for j in range(2):
            sum2[2 * b + j, b] = 1.0

    consts = [jnp.asarray(v) for v in
              (pmat, ssel, posg, posu, pbmat, g1, g2, t2, sum2)]

    y2 = pl.pallas_call(
        functools.partial(_fused_kernel, n_rows=N, blk_r=R, ncols=nc,
                          cin=C, emb_dim=D),
        grid=(ng,),
        in_specs=[
            pl.BlockSpec((R, D), lambda i: (i, 0)),
            pl.BlockSpec((N, D), lambda i: (0, 0)),
            pl.BlockSpec((B, N, C), lambda i: (0, 0, 0)),
            pl.BlockSpec((D * 2 * Cw, 4), lambda i: (0, 0)),
            pl.BlockSpec((D * 2 * Cw, 2), lambda i: (0, 0)),
            pl.BlockSpec((D, 4), lambda i: (0, 0)),
            pl.BlockSpec((D, 2), lambda i: (0, 0)),
            pl.BlockSpec((1, 2), lambda i: (0, 0)),
            pl.BlockSpec((1, 1), lambda i: (0, 0)),
            pl.BlockSpec((D, nc), lambda i: (0, 0)),
            pl.BlockSpec((nc, 4 * B), lambda i: (0, 0)),
            pl.BlockSpec((D, 4, 24), lambda i: (0, 0, 0)),
            pl.BlockSpec((D, 2, 24), lambda i: (0, 0, 0)),
            pl.BlockSpec((B, 24, nc), lambda i: (0, 0, 0)),
            pl.BlockSpec((4, 4 * B), lambda i: (0, 0)),
            pl.BlockSpec((2, 4 * B), lambda i: (0, 0)),
            pl.BlockSpec((2, 2 * B), lambda i: (0, 0)),
            pl.BlockSpec((2 * B, B), lambda i: (0, 0)),
        ],
        out_specs=pl.BlockSpec((B, R), lambda i: (0, i)),
        out_shape=jax.ShapeDtypeStruct((B, N), jnp.float32),
        scratch_shapes=[
            pltpu.VMEM((ng * R, nc), jnp.float32),
            pltpu.VMEM((N, 128), jnp.float32),
        ],
        compiler_params=pltpu.CompilerParams(
            dimension_semantics=("arbitrary",),
        ),
    )(e, e, x, gflat, uflat, gate_bias_pool, update_bias_pool,
      linear_w, lb2, *consts)

    return y2[:, :, None]


# speedup vs baseline: 1.0682x; 1.0003x over previous
import functools

import jax
import jax.numpy as jnp
import numpy as np
from jax.experimental import pallas as pl
from jax.experimental.pallas import tpu as pltpu


def _fused_kernel(e_blk, eall_ref, x_ref, gf_ref, uf_ref, gbp_ref, ubp_ref,
                  lw_ref, lb_ref, pmat_ref, ssel_ref, posg_ref, posu_ref,
                  pb_ref, g1_ref, g2_ref, t2_ref, sum2_ref, out_ref,
                  u_ref, v_ref, *, n_rows, blk_r, ncols, cin, emb_dim):
    i = pl.program_id(0)

    @pl.when(i == 0)
    def _prep():
        cw = gf_ref.shape[0] // (2 * emb_dim)
        w_k = []
        for k in range(2):
            acc = None
            for d in range(emb_dim):
                off = (2 * d + k) * cw
                term = (jnp.dot(gf_ref[off:off + cin, :], posg_ref[d],
                                preferred_element_type=jnp.float32)
                        + jnp.dot(uf_ref[off:off + cin, :], posu_ref[d],
                                  preferred_element_type=jnp.float32))
                acc = term if acc is None else acc + term
            w_k.append(acc)
        nb = x_ref.shape[0]
        u_acc = None
        v_acc = None
        for b in range(nb):
            wab = jnp.dot(w_k[0], pb_ref[b], preferred_element_type=jnp.float32)
            wbb = jnp.dot(w_k[1], pb_ref[b], preferred_element_type=jnp.float32)
            du = jnp.dot(x_ref[b], wab, preferred_element_type=jnp.float32)
            dv = jnp.dot(x_ref[b], wbb, preferred_element_type=jnp.float32)
            u_acc = du if u_acc is None else u_acc + du
            v_acc = dv if v_acc is None else v_acc + dv
        u_ref[0:n_rows, :] = u_acc
        v_ref[:, 0:ncols] = v_acc
        v_ref[:, ncols:ncols + 1] = jnp.ones((n_rows, 1), jnp.float32)

    eb = e_blk[...]
    a = jax.lax.dot_general(eb, eall_ref[...],
                            (((1,), (1,)), ((), ())),
                            preferred_element_type=jnp.float32)
    p = jnp.exp(jnp.minimum(jnp.maximum(a, 0.0), 85.0))
    pv = jnp.dot(p, v_ref[...], preferred_element_type=jnp.float32)
    inv = 1.0 / pv[:, ncols:ncols + 1]

    t = u_ref[pl.ds(i * blk_r, blk_r), :] + pv[:, 0:ncols] * inv
    emul = jnp.dot(eb, pmat_ref[...], preferred_element_type=jnp.float32)
    gu = (jnp.dot(t * emul, ssel_ref[...], preferred_element_type=jnp.float32)
          + jnp.dot(jnp.dot(eb, gbp_ref[...],
                            preferred_element_type=jnp.float32),
                    g1_ref[...], preferred_element_type=jnp.float32)
          + jnp.dot(jnp.dot(eb, ubp_ref[...],
                            preferred_element_type=jnp.float32),
                    g2_ref[...], preferred_element_type=jnp.float32))
    r = jax.nn.sigmoid(gu[:, 0:8])
    h = jnp.tanh(gu[:, 8:16])
    y = jnp.maximum((1.0 - r) * h, 0.0)
    lwt = jnp.dot(lw_ref[...], t2_ref[...],
                  preferred_element_type=jnp.float32)
    yo = (jnp.dot(y * lwt, sum2_ref[...], preferred_element_type=jnp.float32)
          + lb_ref[0:1, 0:1])
    out_ref[...] = yo.T


def kernel(x, e, gate_weights_pool, gate_bias_pool, update_weights_pool,
           update_bias_pool, linear_w, linear_b):
    B, N, C = x.shape
    D = e.shape[1]
    Cw = gate_weights_pool.shape[2]
    R = 1024
    ng = pl.cdiv(N, R)
    nc = 24 * B

    gflat = gate_weights_pool.reshape(D * 2 * Cw, 4)
    uflat = update_weights_pool.reshape(D * 2 * Cw, 2)
    lb2 = linear_b.reshape(1, 1)

    posg = np.zeros((D, 4, 24), np.float32)
    posu = np.zeros((D, 2, 24), np.float32)
    for d in range(D):
        for o in range(4):
            posg[d, o, 4 * d + o] = 1.0
        for o in range(2):
            posu[d, o, 16 + 2 * d + o] = 1.0
    pbmat = np.zeros((B, 24, nc), np.float32)
    for b in range(B):
        for c in range(24):
            pbmat[b, c, 24 * b + c] = 1.0
    pm1 = np.zeros((D, 24), np.float32)
    for d in range(D):
        pm1[d, 4 * d:4 * d + 4] = 1.0
        pm1[d, 16 + 2 * d:16 + 2 * d + 2] = 1.0
    pmat = np.tile(pm1, (1, B))
    ss1 = np.zeros((24, 16), np.float32)
    for d in range(D):
        for j in range(2):
            ss1[4 * d + 2 + j, j] = 1.0
            ss1[16 + 2 * d + j, 8 + j] = 1.0
    ssel = np.zeros((nc, 16), np.float32)
    for b in range(B):
        ssel[b * 24:(b + 1) * 24, 2 * b:2 * b + 2] = ss1[:, 0:2]
        ssel[b * 24:(b + 1) * 24, 8 + 2 * b:8 + 2 * b + 2] = ss1[:, 8:10]
    g1 = np.zeros((4, 16), np.float32)
    g2 = np.zeros((2, 16), np.float32)
    for b in range(B):
        for j in range(2):
            g1[2 + j, 2 * b + j] = 1.0
            g2[j, 8 + 2 * b + j] = 1.0
    t2 = np.zeros((2, 2 * B), np.float32)
    for b in range(B):
        for j in range(2):
            t2[j, 2 * b + j] = 1.0
    sum2 = np.zeros((2 * B, B), np.float32)
    for b in range(B):
        for j in range(2):
            sum2[2 * b + j, b] = 1.0

    consts = [jnp.asarray(v) for v in
              (pmat, ssel, posg, posu, pbmat, g1, g2, t2, sum2)]

    y2 = pl.pallas_call(
        functools.partial(_fused_kernel, n_rows=N, blk_r=R, ncols=nc,
                          cin=C, emb_dim=D),
        grid=(ng,),
        in_specs=[
            pl.BlockSpec((R, D), lambda i: (i, 0)),
            pl.BlockSpec((N, D), lambda i: (0, 0)),
            pl.BlockSpec((B, N, C), lambda i: (0, 0, 0)),
            pl.BlockSpec((D * 2 * Cw, 4), lambda i: (0, 0)),
            pl.BlockSpec((D * 2 * Cw, 2), lambda i: (0, 0)),
            pl.BlockSpec((D, 4), lambda i: (0, 0)),
            pl.BlockSpec((D, 2), lambda i: (0, 0)),
            pl.BlockSpec((1, 2), lambda i: (0, 0)),
            pl.BlockSpec((1, 1), lambda i: (0, 0)),
            pl.BlockSpec((D, nc), lambda i: (0, 0)),
            pl.BlockSpec((nc, 4 * B), lambda i: (0, 0)),
            pl.BlockSpec((D, 4, 24), lambda i: (0, 0, 0)),
            pl.BlockSpec((D, 2, 24), lambda i: (0, 0, 0)),
            pl.BlockSpec((B, 24, nc), lambda i: (0, 0, 0)),
            pl.BlockSpec((4, 4 * B), lambda i: (0, 0)),
            pl.BlockSpec((2, 4 * B), lambda i: (0, 0)),
            pl.BlockSpec((2, 2 * B), lambda i: (0, 0)),
            pl.BlockSpec((2 * B, B), lambda i: (0, 0)),
        ],
        out_specs=pl.BlockSpec((B, R), lambda i: (0, i)),
        out_shape=jax.ShapeDtypeStruct((B, N), jnp.float32),
        scratch_shapes=[
            pltpu.VMEM((ng * R, nc), jnp.float32),
            pltpu.VMEM((N, 128), jnp.float32),
        ],
        compiler_params=pltpu.CompilerParams(
            dimension_semantics=("arbitrary",),
        ),
    )(e, e, x, gflat, uflat, gate_bias_pool, update_bias_pool,
      linear_w, lb2, *consts)

    return y2[:, :, None]


# speedup vs baseline: 1.1262x; 1.0544x over previous
import functools

import jax
import jax.numpy as jnp
import numpy as np
from jax.experimental import pallas as pl
from jax.experimental.pallas import tpu as pltpu


def _fused_kernel(eall_ref, x_ref, gf_ref, uf_ref, gbp_ref, ubp_ref,
                  lw_ref, lb_ref, pmat_ref, ssel_ref, posg_ref, posu_ref,
                  pb_ref, g1_ref, g2_ref, t2_ref, sum2_ref, out_ref,
                  *, n_rows, ncols, cin, emb_dim, chunk):
    cw = gf_ref.shape[0] // (2 * emb_dim)
    w_k = []
    for k in range(2):
        acc = None
        for d in range(emb_dim):
            off = (2 * d + k) * cw
            term = (jnp.dot(gf_ref[off:off + cin, :], posg_ref[d],
                            preferred_element_type=jnp.float32)
                    + jnp.dot(uf_ref[off:off + cin, :], posu_ref[d],
                              preferred_element_type=jnp.float32))
            acc = term if acc is None else acc + term
        w_k.append(acc)
    nb = x_ref.shape[0]
    u = None
    v = None
    for b in range(nb):
        wab = jnp.dot(w_k[0], pb_ref[b], preferred_element_type=jnp.float32)
        wbb = jnp.dot(w_k[1], pb_ref[b], preferred_element_type=jnp.float32)
        du = jnp.dot(x_ref[b], wab, preferred_element_type=jnp.float32)
        dv = jnp.dot(x_ref[b], wbb, preferred_element_type=jnp.float32)
        u = du if u is None else u + du
        v = dv if v is None else v + dv
    va = jnp.concatenate([v, jnp.ones((n_rows, 1), jnp.float32)], axis=1)

    ea = eall_ref[...]
    pv = None
    for c0 in range(0, n_rows, chunk):
        w = min(chunk, n_rows - c0)
        ec = eall_ref[c0:c0 + w, :]
        a = jax.lax.dot_general(ea, ec, (((1,), (1,)), ((), ())),
                                preferred_element_type=jnp.float32)
        p = jnp.exp(jnp.minimum(jnp.maximum(a, 0.0), 85.0))
        term = jnp.dot(p, va[c0:c0 + w, :], preferred_element_type=jnp.float32)
        pv = term if pv is None else pv + term
    inv = 1.0 / pv[:, ncols:ncols + 1]

    t = u + pv[:, 0:ncols] * inv
    emul = jnp.dot(ea, pmat_ref[...], preferred_element_type=jnp.float32)
    gu = (jnp.dot(t * emul, ssel_ref[...], preferred_element_type=jnp.float32)
          + jnp.dot(jnp.dot(ea, gbp_ref[...],
                            preferred_element_type=jnp.float32),
                    g1_ref[...], preferred_element_type=jnp.float32)
          + jnp.dot(jnp.dot(ea, ubp_ref[...],
                            preferred_element_type=jnp.float32),
                    g2_ref[...], preferred_element_type=jnp.float32))
    r = jax.nn.sigmoid(gu[:, 0:8])
    h = jnp.tanh(gu[:, 8:16])
    y = jnp.maximum((1.0 - r) * h, 0.0)
    lwt = jnp.dot(lw_ref[...], t2_ref[...],
                  preferred_element_type=jnp.float32)
    yo = (jnp.dot(y * lwt, sum2_ref[...], preferred_element_type=jnp.float32)
          + lb_ref[0:1, 0:1])
    out_ref[...] = yo.T


def kernel(x, e, gate_weights_pool, gate_bias_pool, update_weights_pool,
           update_bias_pool, linear_w, linear_b):
    B, N, C = x.shape
    D = e.shape[1]
    Cw = gate_weights_pool.shape[2]
    nc = 24 * B

    gflat = gate_weights_pool.reshape(D * 2 * Cw, 4)
    uflat = update_weights_pool.reshape(D * 2 * Cw, 2)
    lb2 = linear_b.reshape(1, 1)

    posg = np.zeros((D, 4, 24), np.float32)
    posu = np.zeros((D, 2, 24), np.float32)
    for d in range(D):
        for o in range(4):
            posg[d, o, 4 * d + o] = 1.0
        for o in range(2):
            posu[d, o, 16 + 2 * d + o] = 1.0
    pbmat = np.zeros((B, 24, nc), np.float32)
    for b in range(B):
        for c in range(24):
            pbmat[b, c, 24 * b + c] = 1.0
    pm1 = np.zeros((D, 24), np.float32)
    for d in range(D):
        pm1[d, 4 * d:4 * d + 4] = 1.0
        pm1[d, 16 + 2 * d:16 + 2 * d + 2] = 1.0
    pmat = np.tile(pm1, (1, B))
    ss1 = np.zeros((24, 16), np.float32)
    for d in range(D):
        for j in range(2):
            ss1[4 * d + 2 + j, j] = 1.0
            ss1[16 + 2 * d + j, 8 + j] = 1.0
    ssel = np.zeros((nc, 16), np.float32)
    for b in range(B):
        ssel[b * 24:(b + 1) * 24, 2 * b:2 * b + 2] = ss1[:, 0:2]
        ssel[b * 24:(b + 1) * 24, 8 + 2 * b:8 + 2 * b + 2] = ss1[:, 8:10]
    g1 = np.zeros((4, 16), np.float32)
    g2 = np.zeros((2, 16), np.float32)
    for b in range(B):
        for j in range(2):
            g1[2 + j, 2 * b + j] = 1.0
            g2[j, 8 + 2 * b + j] = 1.0
    t2 = np.zeros((2, 2 * B), np.float32)
    for b in range(B):
        for j in range(2):
            t2[j, 2 * b + j] = 1.0
    sum2 = np.zeros((2 * B, B), np.float32)
    for b in range(B):
        for j in range(2):
            sum2[2 * b + j, b] = 1.0

    consts = [jnp.asarray(arr) for arr in
              (pmat, ssel, posg, posu, pbmat, g1, g2, t2, sum2)]

    full = lambda *shape: pl.BlockSpec(shape, lambda: tuple(0 for _ in shape))
    y2 = pl.pallas_call(
        functools.partial(_fused_kernel, n_rows=N, ncols=nc, cin=C,
                          emb_dim=D, chunk=512),
        in_specs=[
            full(N, D),
            full(B, N, C),
            full(D * 2 * Cw, 4),
            full(D * 2 * Cw, 2),
            full(D, 4),
            full(D, 2),
            full(1, 2),
            full(1, 1),
            full(D, nc),
            full(nc, 4 * B),
            full(D, 4, 24),
            full(D, 2, 24),
            full(B, 24, nc),
            full(4, 4 * B),
            full(2, 4 * B),
            full(2, 2 * B),
            full(2 * B, B),
        ],
        out_specs=full(B, N),
        out_shape=jax.ShapeDtypeStruct((B, N), jnp.float32),
        compiler_params=pltpu.CompilerParams(
            dimension_semantics=(),
        ),
    )(e, x, gflat, uflat, gate_bias_pool, update_bias_pool,
      linear_w, lb2, *consts)

    return y2[:, :, None]
